# baseline (speedup 1.0000x reference)
import functools

import jax
import jax.numpy as jnp
from jax.experimental import pallas as pl
from jax.experimental.pallas import tpu as pltpu


def _gelu(x):
    c = 0.7978845608028654
    return 0.5 * x * (1.0 + jnp.tanh(c * (x + 0.044715 * x * x * x)))


def _linear_body(x_ref, w_ref, b_ref, o_ref, *, act):
    acc = jnp.dot(x_ref[...], w_ref[...], preferred_element_type=jnp.float32)
    acc = acc + b_ref[...]
    if act:
        acc = _gelu(acc)
    o_ref[...] = acc


def _fused_linear(x, W, b, act=False, bm=512, bn=1024):
    M, K = x.shape
    N = W.shape[1]
    bn = min(bn, N)
    Mp = ((M + bm - 1) // bm) * bm
    xp = jnp.pad(x, ((0, Mp - M), (0, 0))) if Mp != M else x
    b2 = b.reshape(1, N)
    out = pl.pallas_call(
        functools.partial(_linear_body, act=act),
        grid=(Mp // bm, N // bn),
        in_specs=[
            pl.BlockSpec((bm, K), lambda i, j: (i, 0)),
            pl.BlockSpec((K, bn), lambda i, j: (0, j)),
            pl.BlockSpec((1, bn), lambda i, j: (0, j)),
        ],
        out_specs=pl.BlockSpec((bm, bn), lambda i, j: (i, j)),
        out_shape=jax.ShapeDtypeStruct((Mp, N), jnp.float32),
    )(xp, W, b2)
    return out[:M] if Mp != M else out


def _ln(x, g, b):
    mu = x.mean(-1, keepdims=True)
    var = x.var(-1, keepdims=True)
    return (x - mu) / jnp.sqrt(var + 1e-5) * g + b


def kernel(atom_types, edge_index, edge_types, batch_idx, params):
    p = params
    x = _fused_linear(p['atom_emb'][atom_types], p['in_W'], p['in_b'])
    ea = p['edge_emb'][edge_types]
    row, col = edge_index[0], edge_index[1]
    N = x.shape[0]
    for lp in p['mpnn']:
        x_i = x[row]
        x_j = x[col]
        h = _fused_linear(jnp.concatenate([x_i, x_j, ea], -1), lp['m1W'], lp['m1b'], act=True)
        msg = _fused_linear(h, lp['m2W'], lp['m2b'])
        aggr = jax.ops.segment_sum(msg, col, num_segments=N)
        cnt = jax.ops.segment_sum(jnp.ones((col.shape[0], 1), jnp.float32), col, num_segments=N)
        aggr = aggr / jnp.clip(cnt, 1.0, None)
        hu = _fused_linear(jnp.concatenate([x, aggr], -1), lp['u1W'], lp['u1b'], act=True)
        out = _fused_linear(hu, lp['u2W'], lp['u2b'])
        x = _ln(x + out, lp['g'], lp['be'])
    Bn = 2048
    pooled = jax.ops.segment_sum(x, batch_idx, num_segments=Bn)
    cnt = jax.ops.segment_sum(jnp.ones((N, 1), jnp.float32), batch_idx, num_segments=Bn)
    mol = _fused_linear(pooled / jnp.clip(cnt, 1.0, None), p['out_W'], p['out_b'])
    h = _fused_linear(mol, p['proj_W'], p['proj_b'])
    for lp in p['hg']:
        f = _fused_linear(h, lp['f1W'], lp['f1b'], act=True)
        ffn = _fused_linear(f, lp['f2W'], lp['f2b'])
        h = _ln(h + ffn, lp['g'], lp['be'])
    prod = _fused_linear(_fused_linear(h, p['pp1W'], p['pp1b'], act=True), p['pp2W'], p['pp2b'])
    co = _fused_linear(_fused_linear(h, p['cp1W'], p['cp1b'], act=True), p['cp2W'], p['cp2b'])
    return (prod, co)


# speedup vs baseline: 1.0047x; 1.0047x over previous
import functools

import jax
import jax.numpy as jnp
from jax.experimental import pallas as pl
from jax.experimental.pallas import tpu as pltpu


def _gelu(x):
    c = 0.7978845608028654
    return 0.5 * x * (1.0 + jnp.tanh(c * (x + 0.044715 * x * x * x)))


def _linear_body(x_ref, w_ref, b_ref, o_ref, *, act):
    acc = jnp.dot(x_ref[...], w_ref[...], preferred_element_type=jnp.float32)
    acc = acc + b_ref[...]
    if act:
        acc = _gelu(acc)
    o_ref[...] = acc


def _fused_linear(x, W, b, act=False, bm=512, bn=1024):
    M, K = x.shape
    N = W.shape[1]
    bn = min(bn, N)
    Mp = ((M + bm - 1) // bm) * bm
    xp = jnp.pad(x, ((0, Mp - M), (0, 0))) if Mp != M else x
    b2 = b.reshape(1, N)
    out = pl.pallas_call(
        functools.partial(_linear_body, act=act),
        grid=(Mp // bm, N // bn),
        in_specs=[
            pl.BlockSpec((bm, K), lambda i, j: (i, 0)),
            pl.BlockSpec((K, bn), lambda i, j: (0, j)),
            pl.BlockSpec((1, bn), lambda i, j: (0, j)),
        ],
        out_specs=pl.BlockSpec((bm, bn), lambda i, j: (i, j)),
        out_shape=jax.ShapeDtypeStruct((Mp, N), jnp.float32),
    )(xp, W, b2)
    return out[:M] if Mp != M else out


def _linear_ln_body(h_ref, w_ref, b_ref, r_ref, g_ref, be_ref, o_ref):
    acc = jnp.dot(h_ref[...], w_ref[...], preferred_element_type=jnp.float32)
    acc = acc + b_ref[...] + r_ref[...]
    mu = acc.mean(-1, keepdims=True)
    var = ((acc - mu) ** 2).mean(-1, keepdims=True)
    o_ref[...] = (acc - mu) / jnp.sqrt(var + 1e-5) * g_ref[...] + be_ref[...]


def _fused_linear_res_ln(h, W, b, res, g, be, bm=512):
    M, K = h.shape
    N = W.shape[1]
    Mp = ((M + bm - 1) // bm) * bm
    if Mp != M:
        h = jnp.pad(h, ((0, Mp - M), (0, 0)))
        res = jnp.pad(res, ((0, Mp - M), (0, 0)))
    out = pl.pallas_call(
        _linear_ln_body,
        grid=(Mp // bm,),
        in_specs=[
            pl.BlockSpec((bm, K), lambda i: (i, 0)),
            pl.BlockSpec((K, N), lambda i: (0, 0)),
            pl.BlockSpec((1, N), lambda i: (0, 0)),
            pl.BlockSpec((bm, N), lambda i: (i, 0)),
            pl.BlockSpec((1, N), lambda i: (0, 0)),
            pl.BlockSpec((1, N), lambda i: (0, 0)),
        ],
        out_specs=pl.BlockSpec((bm, N), lambda i: (i, 0)),
        out_shape=jax.ShapeDtypeStruct((Mp, N), jnp.float32),
    )(h, W, b.reshape(1, N), res, g.reshape(1, N), be.reshape(1, N))
    return out[:M] if Mp != M else out


def kernel(atom_types, edge_index, edge_types, batch_idx, params):
    p = params
    x = _fused_linear(p['atom_emb'][atom_types], p['in_W'], p['in_b'])
    row, col = edge_index[0], edge_index[1]
    N = x.shape[0]
    E = row.shape[0]

    cnt = jax.ops.segment_sum(jnp.ones((E,), jnp.float32), col, num_segments=N)
    inv_cnt = (1.0 / jnp.clip(cnt, 1.0, None))[:, None]
    has_edge = (cnt > 0.0).astype(jnp.float32)[:, None]

    for lp in p['mpnn']:
        Wab = jnp.concatenate([lp['m1W'][:512], lp['m1W'][512:1024]], axis=1)
        AB = _fused_linear(x, Wab, jnp.zeros((2048,), jnp.float32))
        A, Bm = AB[:, :1024], AB[:, 1024:]
        Ce = p['edge_emb'] @ lp['m1W'][1024:] + lp['m1b']
        pre = A[row] + Bm[col] + Ce[edge_types]
        gmsg = _gelu(pre)
        S = jax.ops.segment_sum(gmsg, col, num_segments=N) * inv_cnt
        aggr = _fused_linear(S, lp['m2W'], jnp.zeros((512,), jnp.float32)) + has_edge * lp['m2b']
        hu = _fused_linear(jnp.concatenate([x, aggr], -1), lp['u1W'], lp['u1b'], act=True)
        x = _fused_linear_res_ln(hu, lp['u2W'], lp['u2b'], x, lp['g'], lp['be'])

    Bn = 2048
    pooled = jax.ops.segment_sum(x, batch_idx, num_segments=Bn)
    pcnt = jax.ops.segment_sum(jnp.ones((N,), jnp.float32), batch_idx, num_segments=Bn)
    mol = _fused_linear(pooled / jnp.clip(pcnt, 1.0, None)[:, None], p['out_W'], p['out_b'])
    h = _fused_linear(mol, p['proj_W'], p['proj_b'])
    for lp in p['hg']:
        f = _fused_linear(h, lp['f1W'], lp['f1b'], act=True)
        h = _fused_linear_res_ln(f, lp['f2W'], lp['f2b'], h, lp['g'], lp['be'])
    prod = _fused_linear(_fused_linear(h, p['pp1W'], p['pp1b'], act=True), p['pp2W'], p['pp2b'])
    co = _fused_linear(_fused_linear(h, p['cp1W'], p['cp1b'], act=True), p['cp2W'], p['cp2b'])
    return (prod, co)


# speedup vs baseline: 1.0070x; 1.0023x over previous
import functools

import jax
import jax.numpy as jnp
from jax.experimental import pallas as pl
from jax.experimental.pallas import tpu as pltpu


def _gelu(x):
    c = 0.7978845608028654
    return 0.5 * x * (1.0 + jnp.tanh(c * (x + 0.044715 * x * x * x)))


def _linear_body(x_ref, w_ref, b_ref, o_ref, *, act):
    acc = jnp.dot(x_ref[...], w_ref[...], preferred_element_type=jnp.float32)
    acc = acc + b_ref[...]
    if act:
        acc = _gelu(acc)
    o_ref[...] = acc


def _fused_linear(x, W, b, act=False, bm=512, bn=1024):
    M, K = x.shape
    N = W.shape[1]
    bn = min(bn, N)
    Mp = ((M + bm - 1) // bm) * bm
    xp = jnp.pad(x, ((0, Mp - M), (0, 0))) if Mp != M else x
    b2 = b.reshape(1, N)
    out = pl.pallas_call(
        functools.partial(_linear_body, act=act),
        grid=(Mp // bm, N // bn),
        in_specs=[
            pl.BlockSpec((bm, K), lambda i, j: (i, 0)),
            pl.BlockSpec((K, bn), lambda i, j: (0, j)),
            pl.BlockSpec((1, bn), lambda i, j: (0, j)),
        ],
        out_specs=pl.BlockSpec((bm, bn), lambda i, j: (i, j)),
        out_shape=jax.ShapeDtypeStruct((Mp, N), jnp.float32),
    )(xp, W, b2)
    return out[:M] if Mp != M else out


def _linear_ln_body(h_ref, w_ref, b_ref, r_ref, g_ref, be_ref, o_ref):
    acc = jnp.dot(h_ref[...], w_ref[...], preferred_element_type=jnp.float32)
    acc = acc + b_ref[...] + r_ref[...]
    mu = acc.mean(-1, keepdims=True)
    var = ((acc - mu) ** 2).mean(-1, keepdims=True)
    o_ref[...] = (acc - mu) / jnp.sqrt(var + 1e-5) * g_ref[...] + be_ref[...]


def _fused_linear_res_ln(h, W, b, res, g, be, bm=512):
    M, K = h.shape
    N = W.shape[1]
    Mp = ((M + bm - 1) // bm) * bm
    if Mp != M:
        h = jnp.pad(h, ((0, Mp - M), (0, 0)))
        res = jnp.pad(res, ((0, Mp - M), (0, 0)))
    out = pl.pallas_call(
        _linear_ln_body,
        grid=(Mp // bm,),
        in_specs=[
            pl.BlockSpec((bm, K), lambda i: (i, 0)),
            pl.BlockSpec((K, N), lambda i: (0, 0)),
            pl.BlockSpec((1, N), lambda i: (0, 0)),
            pl.BlockSpec((bm, N), lambda i: (i, 0)),
            pl.BlockSpec((1, N), lambda i: (0, 0)),
            pl.BlockSpec((1, N), lambda i: (0, 0)),
        ],
        out_specs=pl.BlockSpec((bm, N), lambda i: (i, 0)),
        out_shape=jax.ShapeDtypeStruct((Mp, N), jnp.float32),
    )(h, W, b.reshape(1, N), res, g.reshape(1, N), be.reshape(1, N))
    return out[:M] if Mp != M else out


def kernel(atom_types, edge_index, edge_types, batch_idx, params):
    p = params
    x = _fused_linear(p['atom_emb'][atom_types], p['in_W'], p['in_b'])
    row, col = edge_index[0], edge_index[1]
    N = x.shape[0]
    E = row.shape[0]

    perm = jnp.argsort(col)
    rowp, colp, etp = row[perm], col[perm], edge_types[perm]

    cnt = jax.ops.segment_sum(jnp.ones((E,), jnp.float32), colp, num_segments=N,
                              indices_are_sorted=True)
    inv_cnt = (1.0 / jnp.clip(cnt, 1.0, None))[:, None]
    has_edge = (cnt > 0.0).astype(jnp.float32)[:, None]

    for lp in p['mpnn']:
        Wab = jnp.concatenate([lp['m1W'][:512], lp['m1W'][512:1024]], axis=1)
        AB = _fused_linear(x, Wab, jnp.zeros((2048,), jnp.float32))
        A, Bm = AB[:, :1024], AB[:, 1024:]
        Ce = p['edge_emb'] @ lp['m1W'][1024:] + lp['m1b']
        pre = A[rowp] + Bm[colp] + Ce[etp]
        gmsg = _gelu(pre)
        S = jax.ops.segment_sum(gmsg, colp, num_segments=N,
                                indices_are_sorted=True) * inv_cnt
        aggr = _fused_linear(S, lp['m2W'], jnp.zeros((512,), jnp.float32)) + has_edge * lp['m2b']
        hu = _fused_linear(jnp.concatenate([x, aggr], -1), lp['u1W'], lp['u1b'], act=True)
        x = _fused_linear_res_ln(hu, lp['u2W'], lp['u2b'], x, lp['g'], lp['be'])

    Bn = 2048
    pooled = jax.ops.segment_sum(x, batch_idx, num_segments=Bn)
    pcnt = jax.ops.segment_sum(jnp.ones((N,), jnp.float32), batch_idx, num_segments=Bn)
    mol = _fused_linear(pooled / jnp.clip(pcnt, 1.0, None)[:, None], p['out_W'], p['out_b'])
    h = _fused_linear(mol, p['proj_W'], p['proj_b'])
    for lp in p['hg']:
        f = _fused_linear(h, lp['f1W'], lp['f1b'], act=True)
        h = _fused_linear_res_ln(f, lp['f2W'], lp['f2b'], h, lp['g'], lp['be'])
    prod = _fused_linear(_fused_linear(h, p['pp1W'], p['pp1b'], act=True), p['pp2W'], p['pp2b'])
    co = _fused_linear(_fused_linear(h, p['cp1W'], p['cp1b'], act=True), p['cp2W'], p['cp2b'])
    return (prod, co)
